# Initial kernel scaffold; baseline (speedup 1.0000x reference)
import jax
import jax.numpy as jnp
from jax.experimental import pallas as pl


def kernel(x, edge_index, batch, W1, b1, W2, b2):
    raise NotImplementedError("write your pallas kernel here")


# speedup vs baseline: 52.8195x; 52.8195x over previous
import functools

import jax
import jax.numpy as jnp
from jax import lax
from jax.experimental import pallas as pl
from jax.experimental.pallas import tpu as pltpu
from jax.experimental.pallas import tpu_sc as plsc

N = 100000
E = 1600000
G = 128
H1 = 16

NC = 2
NS = 16
L = 16

NP = 102400
EP = 1605632
ROWS = EP // 128
RPT = ROWS // (NC * NS)
NPT16 = NP // NS
NCH16 = NPT16 // 128
NPT32 = NP // (NC * NS)
NCH32 = NPT32 // 128
BB = 56
NBLK = RPT // BB

_f32 = jnp.float32
_i32 = jnp.int32

_mesh = plsc.VectorSubcoreMesh(core_axis_name="c", subcore_axis_name="s")


def _rsqrt16(d):
    bits = lax.bitcast_convert_type(d, _i32)
    bits = jnp.int32(0x5F3759DF) - lax.shift_right_logical(bits, 1)
    y = lax.bitcast_convert_type(bits, _f32)
    for _ in range(3):
        y = y * (1.5 - 0.5 * d * y * y)
    return y


def _fill(ref, n, value, dtype=_f32):
    for i in range(n // L):
        ref[pl.ds(i * L, L)] = jnp.full((L,), value, dtype)


@functools.partial(
    pl.kernel,
    out_type=[jax.ShapeDtypeStruct((NP,), _f32) for _ in range(2)],
    mesh=_mesh,
    scratch_types=[
        pltpu.VMEM_SHARED((NP,), _f32),
        pltpu.VMEM((BB, 128), _i32),
        pltpu.VMEM((128,), _f32),
        pltpu.VMEM((128,), _f32),
    ],
)
def _k_deg(dst_h, o0, o1, acc_sp, dbuf, onesb, zb):
    c = lax.axis_index("c")
    s = lax.axis_index("s")
    wid = c * NS + s
    _fill(onesb, 128, 1.0)
    _fill(zb, 128, 0.0)

    def zloop(k, _):
        pltpu.sync_copy(zb, acc_sp.at[pl.ds(s * NPT16 + k * 128, 128)])
        return 0

    lax.fori_loop(0, NCH16, zloop, 0)
    plsc.subcore_barrier()

    def bloop(bi, _):
        pltpu.sync_copy(dst_h.at[pl.ds(wid * RPT + bi * BB, BB)], dbuf)

        def eloop(j, _):
            pltpu.sync_copy(onesb, acc_sp.at[dbuf.at[j]], add=True)
            return 0

        lax.fori_loop(0, BB, eloop, 0)
        return 0

    lax.fori_loop(0, NBLK, bloop, 0)
    plsc.subcore_barrier()

    @pl.when(c == 0)
    def _():
        pltpu.sync_copy(acc_sp.at[pl.ds(s * NPT16, NPT16)], o0.at[pl.ds(s * NPT16, NPT16)])

    @pl.when(c == 1)
    def _():
        pltpu.sync_copy(acc_sp.at[pl.ds(s * NPT16, NPT16)], o1.at[pl.ds(s * NPT16, NPT16)])


@functools.partial(
    pl.kernel,
    out_type=[jax.ShapeDtypeStruct((NP,), _f32) for _ in range(4)],
    mesh=_mesh,
    scratch_types=[
        pltpu.VMEM_SHARED((NP,), _f32),
        pltpu.VMEM_SHARED((NP,), _f32),
        pltpu.VMEM((BB, 128), _i32),
        pltpu.VMEM((BB, 128), _i32),
        pltpu.VMEM((128,), _f32),
        pltpu.VMEM((128,), _f32),
        pltpu.VMEM((128,), _f32),
        pltpu.VMEM((128,), _f32),
        pltpu.VMEM((128,), _f32),
        pltpu.VMEM((128,), _f32),
        pltpu.VMEM((128,), _f32),
    ],
)
def _k_t1(src_h, dst_h, d0_h, d1_h, x_h, t1o0, t1o1, dinv_o, xd_o,
          xd_sp, acc_sp, sslab, dslab, zb, gb, d0b, d1b, xb, dvb, xdb):
    c = lax.axis_index("c")
    s = lax.axis_index("s")
    wid = c * NS + s
    _fill(zb, 128, 0.0)

    def nloop(k, _):
        base = s * NPT16 + k * 128
        pltpu.sync_copy(d0_h.at[pl.ds(base, 128)], d0b)
        pltpu.sync_copy(d1_h.at[pl.ds(base, 128)], d1b)
        pltpu.sync_copy(x_h.at[pl.ds(base, 128)], xb)
        for i in range(8):
            sl = pl.ds(i * L, L)
            deg = d0b[sl] + d1b[sl] + 1.0
            dv = _rsqrt16(deg)
            dvb[sl] = dv
            xdb[sl] = xb[sl] * dv
        pltpu.sync_copy(xdb, xd_sp.at[pl.ds(base, 128)])
        pltpu.sync_copy(zb, acc_sp.at[pl.ds(base, 128)])

        @pl.when(c == 0)
        def _():
            pltpu.sync_copy(dvb, dinv_o.at[pl.ds(base, 128)])
            pltpu.sync_copy(xdb, xd_o.at[pl.ds(base, 128)])

        return 0

    lax.fori_loop(0, NCH16, nloop, 0)
    plsc.subcore_barrier()

    def bloop(bi, _):
        pltpu.sync_copy(src_h.at[pl.ds(wid * RPT + bi * BB, BB)], sslab)
        pltpu.sync_copy(dst_h.at[pl.ds(wid * RPT + bi * BB, BB)], dslab)

        def eloop(j, _):
            pltpu.sync_copy(xd_sp.at[sslab.at[j]], gb)
            pltpu.sync_copy(gb, acc_sp.at[dslab.at[j]], add=True)
            return 0

        lax.fori_loop(0, BB, eloop, 0)
        return 0

    lax.fori_loop(0, NBLK, bloop, 0)
    plsc.subcore_barrier()

    @pl.when(c == 0)
    def _():
        pltpu.sync_copy(acc_sp.at[pl.ds(s * NPT16, NPT16)], t1o0.at[pl.ds(s * NPT16, NPT16)])

    @pl.when(c == 1)
    def _():
        pltpu.sync_copy(acc_sp.at[pl.ds(s * NPT16, NPT16)], t1o1.at[pl.ds(s * NPT16, NPT16)])


@functools.partial(
    pl.kernel,
    out_type=[jax.ShapeDtypeStruct((NP,), _f32) for _ in range(6)],
    mesh=_mesh,
    scratch_types=[
        pltpu.VMEM_SHARED((NP,), _f32),
        pltpu.VMEM_SHARED((NP,), _f32),
        pltpu.VMEM_SHARED((NP,), _f32),
        pltpu.VMEM_SHARED((NP,), _f32),
        pltpu.VMEM((BB, 128), _i32),
        pltpu.VMEM((BB, 128), _i32),
        pltpu.VMEM((16, 16), _f32),
        pltpu.VMEM((16, 16), _f32),
        pltpu.VMEM((16, 16), _f32),
        pltpu.VMEM((16, 16), _f32),
        pltpu.VMEM((128,), _f32),
        pltpu.VMEM((128,), _f32),
        pltpu.VMEM((128,), _f32),
        pltpu.VMEM((128,), _f32),
        pltpu.VMEM((128,), _f32),
        pltpu.VMEM((128,), _f32),
        pltpu.VMEM((128,), _f32),
    ],
)
def _k_t2(src_h, dst_h, t10_h, t11_h, dv_h, xd_h, w1_h, b1_h, w20_h, w21_h,
          o00, o01, o10, o11, gd0_o, gd1_o,
          gd0_sp, gd1_sp, a0_sp, a1_sp, sslab, dslab,
          w1v, b1v, w20v, w21v, zb, t0b, t1b, dvb, xdb, g0b, g1b):
    c = lax.axis_index("c")
    s = lax.axis_index("s")
    wid = c * NS + s
    _fill(zb, 128, 0.0)
    pltpu.sync_copy(w1_h, w1v)
    pltpu.sync_copy(b1_h, b1v)
    pltpu.sync_copy(w20_h, w20v)
    pltpu.sync_copy(w21_h, w21v)

    def nloop(k, _):
        base = s * NPT16 + k * 128
        pltpu.sync_copy(t10_h.at[pl.ds(base, 128)], t0b)
        pltpu.sync_copy(t11_h.at[pl.ds(base, 128)], t1b)
        pltpu.sync_copy(dv_h.at[pl.ds(base, 128)], dvb)
        pltpu.sync_copy(xd_h.at[pl.ds(base, 128)], xdb)
        for i in range(8):
            sl = pl.ds(i * L, L)
            dv = dvb[sl]
            s1 = dv * (t0b[sl] + t1b[sl] + xdb[sl])
            acc0 = jnp.zeros((L,), _f32)
            acc1 = jnp.zeros((L,), _f32)
            for j in range(H1):
                hr = jnp.maximum(s1 * w1v[j] + b1v[j], 0.0)
                acc0 = acc0 + hr * w20v[j]
                acc1 = acc1 + hr * w21v[j]
            g0b[sl] = acc0 * dv
            g1b[sl] = acc1 * dv
        pltpu.sync_copy(g0b, gd0_sp.at[pl.ds(base, 128)])
        pltpu.sync_copy(g1b, gd1_sp.at[pl.ds(base, 128)])
        pltpu.sync_copy(zb, a0_sp.at[pl.ds(base, 128)])
        pltpu.sync_copy(zb, a1_sp.at[pl.ds(base, 128)])

        @pl.when(c == 0)
        def _():
            pltpu.sync_copy(g0b, gd0_o.at[pl.ds(base, 128)])
            pltpu.sync_copy(g1b, gd1_o.at[pl.ds(base, 128)])

        return 0

    lax.fori_loop(0, NCH16, nloop, 0)
    plsc.subcore_barrier()

    def bloop(bi, _):
        pltpu.sync_copy(src_h.at[pl.ds(wid * RPT + bi * BB, BB)], sslab)
        pltpu.sync_copy(dst_h.at[pl.ds(wid * RPT + bi * BB, BB)], dslab)

        def eloop(j, _):
            pltpu.sync_copy(gd0_sp.at[sslab.at[j]], g0b)
            pltpu.sync_copy(g0b, a0_sp.at[dslab.at[j]], add=True)
            pltpu.sync_copy(gd1_sp.at[sslab.at[j]], g1b)
            pltpu.sync_copy(g1b, a1_sp.at[dslab.at[j]], add=True)
            return 0

        lax.fori_loop(0, BB, eloop, 0)
        return 0

    lax.fori_loop(0, NBLK, bloop, 0)
    plsc.subcore_barrier()

    @pl.when(c == 0)
    def _():
        pltpu.sync_copy(a0_sp.at[pl.ds(s * NPT16, NPT16)], o00.at[pl.ds(s * NPT16, NPT16)])
        pltpu.sync_copy(a1_sp.at[pl.ds(s * NPT16, NPT16)], o01.at[pl.ds(s * NPT16, NPT16)])

    @pl.when(c == 1)
    def _():
        pltpu.sync_copy(a0_sp.at[pl.ds(s * NPT16, NPT16)], o10.at[pl.ds(s * NPT16, NPT16)])
        pltpu.sync_copy(a1_sp.at[pl.ds(s * NPT16, NPT16)], o11.at[pl.ds(s * NPT16, NPT16)])


@functools.partial(
    pl.kernel,
    out_type=[jax.ShapeDtypeStruct((128,), _f32) for _ in range(6)],
    mesh=_mesh,
    scratch_types=[
        pltpu.VMEM_SHARED((136,), _f32),
        pltpu.VMEM_SHARED((136,), _f32),
        pltpu.VMEM_SHARED((136,), _f32),
        pltpu.VMEM((128,), _i32),
        pltpu.VMEM((136,), _f32),
        pltpu.VMEM((128,), _f32),
        pltpu.VMEM((128,), _f32),
        pltpu.VMEM((128,), _f32),
        pltpu.VMEM((128,), _f32),
        pltpu.VMEM((128,), _f32),
        pltpu.VMEM((128,), _f32),
        pltpu.VMEM((128,), _f32),
        pltpu.VMEM((128,), _f32),
        pltpu.VMEM((128,), _f32),
        pltpu.VMEM((128,), _f32),
    ],
)
def _k_pool(t00_h, t01_h, t10_h, t11_h, dv_h, g0_h, g1_h, bt_h,
            po00, po01, pc0, po10, po11, pc1,
            ps0, ps1, psc, btb, zb, onesb,
            a00b, a01b, a10b, a11b, dvb, g0b, g1b, o0b, o1b):
    c = lax.axis_index("c")
    s = lax.axis_index("s")
    wid = c * NS + s
    _fill(zb, 136, 0.0)
    _fill(onesb, 128, 1.0)

    @pl.when(s == 0)
    def _():
        pltpu.sync_copy(zb, ps0)
        pltpu.sync_copy(zb, ps1)
        pltpu.sync_copy(zb, psc)

    plsc.subcore_barrier()

    def nloop(k, _):
        base = wid * NPT32 + k * 128
        pltpu.sync_copy(bt_h.at[pl.ds(base, 128)], btb)
        pltpu.sync_copy(t00_h.at[pl.ds(base, 128)], a00b)
        pltpu.sync_copy(t01_h.at[pl.ds(base, 128)], a01b)
        pltpu.sync_copy(t10_h.at[pl.ds(base, 128)], a10b)
        pltpu.sync_copy(t11_h.at[pl.ds(base, 128)], a11b)
        pltpu.sync_copy(dv_h.at[pl.ds(base, 128)], dvb)
        pltpu.sync_copy(g0_h.at[pl.ds(base, 128)], g0b)
        pltpu.sync_copy(g1_h.at[pl.ds(base, 128)], g1b)
        for i in range(8):
            sl = pl.ds(i * L, L)
            dv = dvb[sl]
            o0b[sl] = dv * (a00b[sl] + a10b[sl] + g0b[sl])
            o1b[sl] = dv * (a01b[sl] + a11b[sl] + g1b[sl])
        pltpu.sync_copy(o0b, ps0.at[btb], add=True)
        pltpu.sync_copy(o1b, ps1.at[btb], add=True)
        pltpu.sync_copy(onesb, psc.at[btb], add=True)
        return 0

    lax.fori_loop(0, NCH32, nloop, 0)
    plsc.subcore_barrier()

    @pl.when(jnp.logical_and(s == 0, c == 0))
    def _():
        pltpu.sync_copy(ps0.at[pl.ds(0, 128)], po00)
        pltpu.sync_copy(ps1.at[pl.ds(0, 128)], po01)
        pltpu.sync_copy(psc.at[pl.ds(0, 128)], pc0)

    @pl.when(jnp.logical_and(s == 0, c == 1))
    def _():
        pltpu.sync_copy(ps0.at[pl.ds(0, 128)], po10)
        pltpu.sync_copy(ps1.at[pl.ds(0, 128)], po11)
        pltpu.sync_copy(psc.at[pl.ds(0, 128)], pc1)


def kernel(x, edge_index, batch, W1, b1, W2, b2):
    x_pad = jnp.concatenate([x[:, 0], jnp.zeros((NP - N,), _f32)])
    pad_idx = jnp.full((EP - E,), N, _i32)
    src2d = jnp.concatenate([edge_index[0], pad_idx]).reshape(ROWS, 128)
    dst2d = jnp.concatenate([edge_index[1], pad_idx]).reshape(ROWS, 128)
    bt_pad = jnp.concatenate([batch, jnp.full((NP - N,), G, _i32)])
    w1m = jnp.broadcast_to(W1.reshape(H1)[:, None], (H1, 16)).astype(_f32)
    b1m = jnp.broadcast_to(b1[:, None], (H1, 16)).astype(_f32)
    w20m = jnp.broadcast_to(W2[:, 0][:, None], (H1, 16)).astype(_f32)
    w21m = jnp.broadcast_to(W2[:, 1][:, None], (H1, 16)).astype(_f32)

    d0, d1 = _k_deg(dst2d)
    t10, t11, dinv, xd = _k_t1(src2d, dst2d, d0, d1, x_pad)
    t00, t01, t10_, t11_, gd0, gd1 = _k_t2(
        src2d, dst2d, t10, t11, dinv, xd, w1m, b1m, w20m, w21m)
    p00, p01, c0, p10, p11, c1 = _k_pool(
        t00, t01, t10_, t11_, dinv, gd0, gd1, bt_pad)

    pool = jnp.stack([p00 + p10, p01 + p11], axis=1)
    cnt = c0 + c1
    mean = pool / jnp.maximum(cnt, 1.0)[:, None] + b2[None, :]
    return jax.nn.log_softmax(mean, axis=1)


# speedup vs baseline: 57.3496x; 1.0858x over previous
import functools

import jax
import jax.numpy as jnp
from jax import lax
from jax.experimental import pallas as pl
from jax.experimental.pallas import tpu as pltpu
from jax.experimental.pallas import tpu_sc as plsc

N = 100000
E = 1600000
G = 128
H1 = 16

NC = 2
NS = 16
L = 16

NP = 102400
EP = 1605632
EPT = EP // (NC * NS)
C = 1024
NCK = EPT // C
NPT16 = NP // NS
NCH16 = NPT16 // 128
NPT32 = NP // (NC * NS)
NCH32 = NPT32 // 128

_f32 = jnp.float32
_i32 = jnp.int32

_mesh = plsc.VectorSubcoreMesh(core_axis_name="c", subcore_axis_name="s")


def _rsqrt16(d):
    bits = lax.bitcast_convert_type(d, _i32)
    bits = jnp.int32(0x5F3759DF) - lax.shift_right_logical(bits, 1)
    y = lax.bitcast_convert_type(bits, _f32)
    for _ in range(3):
        y = y * (1.5 - 0.5 * d * y * y)
    return y


def _fill(ref, n, value, dtype=_f32):
    for i in range(n // L):
        ref[pl.ds(i * L, L)] = jnp.full((L,), value, dtype)


@functools.partial(
    pl.kernel,
    out_type=[jax.ShapeDtypeStruct((NP,), _f32) for _ in range(2)],
    mesh=_mesh,
    scratch_types=[
        pltpu.VMEM_SHARED((NP,), _f32),
        pltpu.VMEM((C,), _i32),
        pltpu.VMEM((C,), _f32),
        pltpu.VMEM((128,), _f32),
    ],
)
def _k_deg(dst_h, o0, o1, acc_sp, dbuf, onesb, zb):
    c = lax.axis_index("c")
    s = lax.axis_index("s")
    wid = c * NS + s
    _fill(onesb, C, 1.0)
    _fill(zb, 128, 0.0)

    def zloop(k, _):
        pltpu.sync_copy(zb, acc_sp.at[pl.ds(s * NPT16 + k * 128, 128)])
        return 0

    lax.fori_loop(0, NCH16, zloop, 0)
    plsc.subcore_barrier()

    def eloop(k, _):
        pltpu.sync_copy(dst_h.at[pl.ds(wid * EPT + k * C, C)], dbuf)
        pltpu.sync_copy(onesb, acc_sp.at[dbuf], add=True)
        return 0

    lax.fori_loop(0, NCK, eloop, 0)
    plsc.subcore_barrier()

    @pl.when(c == 0)
    def _():
        pltpu.sync_copy(acc_sp.at[pl.ds(s * NPT16, NPT16)], o0.at[pl.ds(s * NPT16, NPT16)])

    @pl.when(c == 1)
    def _():
        pltpu.sync_copy(acc_sp.at[pl.ds(s * NPT16, NPT16)], o1.at[pl.ds(s * NPT16, NPT16)])


@functools.partial(
    pl.kernel,
    out_type=[jax.ShapeDtypeStruct((NP,), _f32) for _ in range(4)],
    mesh=_mesh,
    scratch_types=[
        pltpu.VMEM_SHARED((NP,), _f32),
        pltpu.VMEM_SHARED((NP,), _f32),
        pltpu.VMEM((C,), _i32),
        pltpu.VMEM((C,), _i32),
        pltpu.VMEM((128,), _f32),
        pltpu.VMEM((C,), _f32),
        pltpu.VMEM((128,), _f32),
        pltpu.VMEM((128,), _f32),
        pltpu.VMEM((128,), _f32),
        pltpu.VMEM((128,), _f32),
        pltpu.VMEM((128,), _f32),
    ],
)
def _k_t1(src_h, dst_h, d0_h, d1_h, x_h, t1o0, t1o1, dinv_o, xd_o,
          xd_sp, acc_sp, sslab, dslab, zb, gb, d0b, d1b, xb, dvb, xdb):
    c = lax.axis_index("c")
    s = lax.axis_index("s")
    wid = c * NS + s
    _fill(zb, 128, 0.0)

    def nloop(k, _):
        base = s * NPT16 + k * 128
        pltpu.sync_copy(d0_h.at[pl.ds(base, 128)], d0b)
        pltpu.sync_copy(d1_h.at[pl.ds(base, 128)], d1b)
        pltpu.sync_copy(x_h.at[pl.ds(base, 128)], xb)
        for i in range(8):
            sl = pl.ds(i * L, L)
            deg = d0b[sl] + d1b[sl] + 1.0
            dv = _rsqrt16(deg)
            dvb[sl] = dv
            xdb[sl] = xb[sl] * dv
        pltpu.sync_copy(xdb, xd_sp.at[pl.ds(base, 128)])
        pltpu.sync_copy(zb, acc_sp.at[pl.ds(base, 128)])

        @pl.when(c == 0)
        def _():
            pltpu.sync_copy(dvb, dinv_o.at[pl.ds(base, 128)])
            pltpu.sync_copy(xdb, xd_o.at[pl.ds(base, 128)])

        return 0

    lax.fori_loop(0, NCH16, nloop, 0)
    plsc.subcore_barrier()

    def eloop(k, _):
        base = wid * EPT + k * C
        pltpu.sync_copy(src_h.at[pl.ds(base, C)], sslab)
        pltpu.sync_copy(dst_h.at[pl.ds(base, C)], dslab)
        pltpu.sync_copy(xd_sp.at[sslab], gb)
        pltpu.sync_copy(gb, acc_sp.at[dslab], add=True)
        return 0

    lax.fori_loop(0, NCK, eloop, 0)
    plsc.subcore_barrier()

    @pl.when(c == 0)
    def _():
        pltpu.sync_copy(acc_sp.at[pl.ds(s * NPT16, NPT16)], t1o0.at[pl.ds(s * NPT16, NPT16)])

    @pl.when(c == 1)
    def _():
        pltpu.sync_copy(acc_sp.at[pl.ds(s * NPT16, NPT16)], t1o1.at[pl.ds(s * NPT16, NPT16)])


@functools.partial(
    pl.kernel,
    out_type=[jax.ShapeDtypeStruct((NP,), _f32) for _ in range(6)],
    mesh=_mesh,
    scratch_types=[
        pltpu.VMEM_SHARED((NP,), _f32),
        pltpu.VMEM_SHARED((NP,), _f32),
        pltpu.VMEM_SHARED((NP,), _f32),
        pltpu.VMEM_SHARED((NP,), _f32),
        pltpu.VMEM((C,), _i32),
        pltpu.VMEM((C,), _i32),
        pltpu.VMEM((C,), _f32),
        pltpu.VMEM((C,), _f32),
        pltpu.VMEM((16, 16), _f32),
        pltpu.VMEM((16, 16), _f32),
        pltpu.VMEM((16, 16), _f32),
        pltpu.VMEM((16, 16), _f32),
        pltpu.VMEM((128,), _f32),
        pltpu.VMEM((128,), _f32),
        pltpu.VMEM((128,), _f32),
        pltpu.VMEM((128,), _f32),
        pltpu.VMEM((128,), _f32),
        pltpu.VMEM((128,), _f32),
        pltpu.VMEM((128,), _f32),
    ],
)
def _k_t2(src_h, dst_h, t10_h, t11_h, dv_h, xd_h, w1_h, b1_h, w20_h, w21_h,
          o00, o01, o10, o11, gd0_o, gd1_o,
          gd0_sp, gd1_sp, a0_sp, a1_sp, sslab, dslab, e0b, e1b,
          w1v, b1v, w20v, w21v, zb, t0b, t1b, dvb, xdb, g0b, g1b):
    c = lax.axis_index("c")
    s = lax.axis_index("s")
    wid = c * NS + s
    _fill(zb, 128, 0.0)
    pltpu.sync_copy(w1_h, w1v)
    pltpu.sync_copy(b1_h, b1v)
    pltpu.sync_copy(w20_h, w20v)
    pltpu.sync_copy(w21_h, w21v)

    def nloop(k, _):
        base = s * NPT16 + k * 128
        pltpu.sync_copy(t10_h.at[pl.ds(base, 128)], t0b)
        pltpu.sync_copy(t11_h.at[pl.ds(base, 128)], t1b)
        pltpu.sync_copy(dv_h.at[pl.ds(base, 128)], dvb)
        pltpu.sync_copy(xd_h.at[pl.ds(base, 128)], xdb)
        for i in range(8):
            sl = pl.ds(i * L, L)
            dv = dvb[sl]
            s1 = dv * (t0b[sl] + t1b[sl] + xdb[sl])
            acc0 = jnp.zeros((L,), _f32)
            acc1 = jnp.zeros((L,), _f32)
            for j in range(H1):
                hr = jnp.maximum(s1 * w1v[j] + b1v[j], 0.0)
                acc0 = acc0 + hr * w20v[j]
                acc1 = acc1 + hr * w21v[j]
            g0b[sl] = acc0 * dv
            g1b[sl] = acc1 * dv
        pltpu.sync_copy(g0b, gd0_sp.at[pl.ds(base, 128)])
        pltpu.sync_copy(g1b, gd1_sp.at[pl.ds(base, 128)])
        pltpu.sync_copy(zb, a0_sp.at[pl.ds(base, 128)])
        pltpu.sync_copy(zb, a1_sp.at[pl.ds(base, 128)])

        @pl.when(c == 0)
        def _():
            pltpu.sync_copy(g0b, gd0_o.at[pl.ds(base, 128)])
            pltpu.sync_copy(g1b, gd1_o.at[pl.ds(base, 128)])

        return 0

    lax.fori_loop(0, NCH16, nloop, 0)
    plsc.subcore_barrier()

    def eloop(k, _):
        base = wid * EPT + k * C
        pltpu.sync_copy(src_h.at[pl.ds(base, C)], sslab)
        pltpu.sync_copy(dst_h.at[pl.ds(base, C)], dslab)
        pltpu.sync_copy(gd0_sp.at[sslab], e0b)
        pltpu.sync_copy(e0b, a0_sp.at[dslab], add=True)
        pltpu.sync_copy(gd1_sp.at[sslab], e1b)
        pltpu.sync_copy(e1b, a1_sp.at[dslab], add=True)
        return 0

    lax.fori_loop(0, NCK, eloop, 0)
    plsc.subcore_barrier()

    @pl.when(c == 0)
    def _():
        pltpu.sync_copy(a0_sp.at[pl.ds(s * NPT16, NPT16)], o00.at[pl.ds(s * NPT16, NPT16)])
        pltpu.sync_copy(a1_sp.at[pl.ds(s * NPT16, NPT16)], o01.at[pl.ds(s * NPT16, NPT16)])

    @pl.when(c == 1)
    def _():
        pltpu.sync_copy(a0_sp.at[pl.ds(s * NPT16, NPT16)], o10.at[pl.ds(s * NPT16, NPT16)])
        pltpu.sync_copy(a1_sp.at[pl.ds(s * NPT16, NPT16)], o11.at[pl.ds(s * NPT16, NPT16)])


@functools.partial(
    pl.kernel,
    out_type=[jax.ShapeDtypeStruct((128,), _f32) for _ in range(6)],
    mesh=_mesh,
    scratch_types=[
        pltpu.VMEM_SHARED((136,), _f32),
        pltpu.VMEM_SHARED((136,), _f32),
        pltpu.VMEM_SHARED((136,), _f32),
        pltpu.VMEM((128,), _i32),
        pltpu.VMEM((136,), _f32),
        pltpu.VMEM((128,), _f32),
        pltpu.VMEM((128,), _f32),
        pltpu.VMEM((128,), _f32),
        pltpu.VMEM((128,), _f32),
        pltpu.VMEM((128,), _f32),
        pltpu.VMEM((128,), _f32),
        pltpu.VMEM((128,), _f32),
        pltpu.VMEM((128,), _f32),
        pltpu.VMEM((128,), _f32),
        pltpu.VMEM((128,), _f32),
    ],
)
def _k_pool(t00_h, t01_h, t10_h, t11_h, dv_h, g0_h, g1_h, bt_h,
            po00, po01, pc0, po10, po11, pc1,
            ps0, ps1, psc, btb, zb, onesb,
            a00b, a01b, a10b, a11b, dvb, g0b, g1b, o0b, o1b):
    c = lax.axis_index("c")
    s = lax.axis_index("s")
    wid = c * NS + s
    _fill(zb, 136, 0.0)
    _fill(onesb, 128, 1.0)

    @pl.when(s == 0)
    def _():
        pltpu.sync_copy(zb, ps0)
        pltpu.sync_copy(zb, ps1)
        pltpu.sync_copy(zb, psc)

    plsc.subcore_barrier()

    def nloop(k, _):
        base = wid * NPT32 + k * 128
        pltpu.sync_copy(bt_h.at[pl.ds(base, 128)], btb)
        pltpu.sync_copy(t00_h.at[pl.ds(base, 128)], a00b)
        pltpu.sync_copy(t01_h.at[pl.ds(base, 128)], a01b)
        pltpu.sync_copy(t10_h.at[pl.ds(base, 128)], a10b)
        pltpu.sync_copy(t11_h.at[pl.ds(base, 128)], a11b)
        pltpu.sync_copy(dv_h.at[pl.ds(base, 128)], dvb)
        pltpu.sync_copy(g0_h.at[pl.ds(base, 128)], g0b)
        pltpu.sync_copy(g1_h.at[pl.ds(base, 128)], g1b)
        for i in range(8):
            sl = pl.ds(i * L, L)
            dv = dvb[sl]
            o0b[sl] = dv * (a00b[sl] + a10b[sl] + g0b[sl])
            o1b[sl] = dv * (a01b[sl] + a11b[sl] + g1b[sl])
        pltpu.sync_copy(o0b, ps0.at[btb], add=True)
        pltpu.sync_copy(o1b, ps1.at[btb], add=True)
        pltpu.sync_copy(onesb, psc.at[btb], add=True)
        return 0

    lax.fori_loop(0, NCH32, nloop, 0)
    plsc.subcore_barrier()

    @pl.when(jnp.logical_and(s == 0, c == 0))
    def _():
        pltpu.sync_copy(ps0.at[pl.ds(0, 128)], po00)
        pltpu.sync_copy(ps1.at[pl.ds(0, 128)], po01)
        pltpu.sync_copy(psc.at[pl.ds(0, 128)], pc0)

    @pl.when(jnp.logical_and(s == 0, c == 1))
    def _():
        pltpu.sync_copy(ps0.at[pl.ds(0, 128)], po10)
        pltpu.sync_copy(ps1.at[pl.ds(0, 128)], po11)
        pltpu.sync_copy(psc.at[pl.ds(0, 128)], pc1)


def kernel(x, edge_index, batch, W1, b1, W2, b2):
    x_pad = jnp.concatenate([x[:, 0], jnp.zeros((NP - N,), _f32)])
    pad_idx = jnp.full((EP - E,), N, _i32)
    src1d = jnp.concatenate([edge_index[0], pad_idx])
    dst1d = jnp.concatenate([edge_index[1], pad_idx])
    bt_pad = jnp.concatenate([batch, jnp.full((NP - N,), G, _i32)])
    w1m = jnp.broadcast_to(W1.reshape(H1)[:, None], (H1, 16)).astype(_f32)
    b1m = jnp.broadcast_to(b1[:, None], (H1, 16)).astype(_f32)
    w20m = jnp.broadcast_to(W2[:, 0][:, None], (H1, 16)).astype(_f32)
    w21m = jnp.broadcast_to(W2[:, 1][:, None], (H1, 16)).astype(_f32)

    d0, d1 = _k_deg(dst1d)
    t10, t11, dinv, xd = _k_t1(src1d, dst1d, d0, d1, x_pad)
    t00, t01, t10_, t11_, gd0, gd1 = _k_t2(
        src1d, dst1d, t10, t11, dinv, xd, w1m, b1m, w20m, w21m)
    p00, p01, c0, p10, p11, c1 = _k_pool(
        t00, t01, t10_, t11_, dinv, gd0, gd1, bt_pad)

    pool = jnp.stack([p00 + p10, p01 + p11], axis=1)
    cnt = c0 + c1
    mean = pool / jnp.maximum(cnt, 1.0)[:, None] + b2[None, :]
    return jax.nn.log_softmax(mean, axis=1)


# speedup vs baseline: 128.4934x; 2.2405x over previous
import functools

import jax
import jax.numpy as jnp
from jax import lax
from jax.experimental import pallas as pl
from jax.experimental.pallas import tpu as pltpu
from jax.experimental.pallas import tpu_sc as plsc

N = 100000
E = 1600000
G = 128
H1 = 16

NC = 2
NS = 16
L = 16

NP = 102400
EP = 1605632
EPT = EP // (NC * NS)
C = 7168
NCK = EPT // C
NPT16 = NP // NS
NPT32 = NP // (NC * NS)
ZB = 3200

_f32 = jnp.float32
_i32 = jnp.int32

_mesh = plsc.VectorSubcoreMesh(core_axis_name="c", subcore_axis_name="s")


def _rsqrt16(d):
    bits = lax.bitcast_convert_type(d, _i32)
    bits = jnp.int32(0x5F3759DF) - lax.shift_right_logical(bits, 1)
    y = lax.bitcast_convert_type(bits, _f32)
    for _ in range(3):
        y = y * (1.5 - 0.5 * d * y * y)
    return y


def _fill(ref, n, value, dtype=_f32):
    for i in range(n // L):
        ref[pl.ds(i * L, L)] = jnp.full((L,), value, dtype)


@functools.partial(
    pl.kernel,
    out_type=[jax.ShapeDtypeStruct((NP,), _f32) for _ in range(2)],
    mesh=_mesh,
    scratch_types=[
        pltpu.VMEM_SHARED((NP,), _f32),
        pltpu.VMEM((C,), _i32),
        pltpu.VMEM((C,), _f32),
        pltpu.VMEM((ZB,), _f32),
    ],
)
def _k_deg(dst_h, o0, o1, acc_sp, dbuf, onesb, zb):
    c = lax.axis_index("c")
    s = lax.axis_index("s")
    wid = c * NS + s

    def fl(i, _):
        onesb[pl.ds(i * L, L)] = jnp.full((L,), 1.0, _f32)
        return 0

    lax.fori_loop(0, C // L, fl, 0)

    def fz(i, _):
        zb[pl.ds(i * L, L)] = jnp.zeros((L,), _f32)
        return 0

    lax.fori_loop(0, ZB // L, fz, 0)
    pltpu.sync_copy(zb, acc_sp.at[pl.ds(s * NPT16, ZB)])
    pltpu.sync_copy(zb, acc_sp.at[pl.ds(s * NPT16 + ZB, ZB)])
    plsc.subcore_barrier()

    def eloop(k, _):
        pltpu.sync_copy(dst_h.at[pl.ds(wid * EPT + k * C, C)], dbuf)
        pltpu.sync_copy(onesb, acc_sp.at[dbuf], add=True)
        return 0

    lax.fori_loop(0, NCK, eloop, 0)
    plsc.subcore_barrier()

    @pl.when(c == 0)
    def _():
        pltpu.sync_copy(acc_sp.at[pl.ds(s * NPT16, NPT16)], o0.at[pl.ds(s * NPT16, NPT16)])

    @pl.when(c == 1)
    def _():
        pltpu.sync_copy(acc_sp.at[pl.ds(s * NPT16, NPT16)], o1.at[pl.ds(s * NPT16, NPT16)])


@functools.partial(
    pl.kernel,
    out_type=[jax.ShapeDtypeStruct((NP,), _f32) for _ in range(4)],
    mesh=_mesh,
    scratch_types=[
        pltpu.VMEM_SHARED((NP,), _f32),
        pltpu.VMEM_SHARED((NP,), _f32),
        pltpu.VMEM((C,), _i32),
        pltpu.VMEM((C,), _i32),
        pltpu.VMEM((ZB,), _f32),
        pltpu.VMEM((C,), _f32),
        pltpu.VMEM((NPT16,), _f32),
        pltpu.VMEM((NPT16,), _f32),
        pltpu.VMEM((NPT16,), _f32),
        pltpu.VMEM((NPT16,), _f32),
        pltpu.VMEM((NPT16,), _f32),
    ],
)
def _k_t1(src_h, dst_h, d0_h, d1_h, x_h, t1o0, t1o1, dinv_o, xd_o,
          xd_sp, acc_sp, sslab, dslab, zb, gb, d0b, d1b, xb, dvb, xdb):
    c = lax.axis_index("c")
    s = lax.axis_index("s")
    wid = c * NS + s
    base = s * NPT16
    pltpu.sync_copy(d0_h.at[pl.ds(base, NPT16)], d0b)
    pltpu.sync_copy(d1_h.at[pl.ds(base, NPT16)], d1b)
    pltpu.sync_copy(x_h.at[pl.ds(base, NPT16)], xb)

    def cloop(i, _):
        sl = pl.ds(i * L, L)
        deg = d0b[sl] + d1b[sl] + 1.0
        dv = _rsqrt16(deg)
        dvb[sl] = dv
        xdb[sl] = xb[sl] * dv
        return 0

    lax.fori_loop(0, NPT16 // L, cloop, 0)
    pltpu.sync_copy(xdb, xd_sp.at[pl.ds(base, NPT16)])

    def fz(i, _):
        zb[pl.ds(i * L, L)] = jnp.zeros((L,), _f32)
        return 0

    lax.fori_loop(0, ZB // L, fz, 0)
    pltpu.sync_copy(zb, acc_sp.at[pl.ds(base, ZB)])
    pltpu.sync_copy(zb, acc_sp.at[pl.ds(base + ZB, ZB)])

    @pl.when(c == 0)
    def _():
        pltpu.sync_copy(dvb, dinv_o.at[pl.ds(base, NPT16)])
        pltpu.sync_copy(xdb, xd_o.at[pl.ds(base, NPT16)])

    plsc.subcore_barrier()

    def eloop(k, _):
        base = wid * EPT + k * C
        pltpu.sync_copy(src_h.at[pl.ds(base, C)], sslab)
        pltpu.sync_copy(dst_h.at[pl.ds(base, C)], dslab)
        pltpu.sync_copy(xd_sp.at[sslab], gb)
        pltpu.sync_copy(gb, acc_sp.at[dslab], add=True)
        return 0

    lax.fori_loop(0, NCK, eloop, 0)
    plsc.subcore_barrier()

    @pl.when(c == 0)
    def _():
        pltpu.sync_copy(acc_sp.at[pl.ds(s * NPT16, NPT16)], t1o0.at[pl.ds(s * NPT16, NPT16)])

    @pl.when(c == 1)
    def _():
        pltpu.sync_copy(acc_sp.at[pl.ds(s * NPT16, NPT16)], t1o1.at[pl.ds(s * NPT16, NPT16)])


@functools.partial(
    pl.kernel,
    out_type=[jax.ShapeDtypeStruct((NP,), _f32) for _ in range(6)],
    mesh=_mesh,
    scratch_types=[
        pltpu.VMEM_SHARED((NP,), _f32),
        pltpu.VMEM_SHARED((NP,), _f32),
        pltpu.VMEM_SHARED((NP,), _f32),
        pltpu.VMEM_SHARED((NP,), _f32),
        pltpu.VMEM((C,), _i32),
        pltpu.VMEM((C,), _i32),
        pltpu.VMEM((C,), _f32),
        pltpu.VMEM((C,), _f32),
        pltpu.VMEM((16, 16), _f32),
        pltpu.VMEM((16, 16), _f32),
        pltpu.VMEM((16, 16), _f32),
        pltpu.VMEM((16, 16), _f32),
        pltpu.VMEM((ZB,), _f32),
        pltpu.VMEM((NPT16,), _f32),
        pltpu.VMEM((NPT16,), _f32),
        pltpu.VMEM((NPT16,), _f32),
        pltpu.VMEM((NPT16,), _f32),
        pltpu.VMEM((NPT16,), _f32),
        pltpu.VMEM((NPT16,), _f32),
    ],
)
def _k_t2(src_h, dst_h, t10_h, t11_h, dv_h, xd_h, w1_h, b1_h, w20_h, w21_h,
          o00, o01, o10, o11, gd0_o, gd1_o,
          gd0_sp, gd1_sp, a0_sp, a1_sp, sslab, dslab, e0b, e1b,
          w1v, b1v, w20v, w21v, zb, t0b, t1b, dvb, xdb, g0b, g1b):
    c = lax.axis_index("c")
    s = lax.axis_index("s")
    wid = c * NS + s
    pltpu.sync_copy(w1_h, w1v)
    pltpu.sync_copy(b1_h, b1v)
    pltpu.sync_copy(w20_h, w20v)
    pltpu.sync_copy(w21_h, w21v)

    base = s * NPT16
    pltpu.sync_copy(t10_h.at[pl.ds(base, NPT16)], t0b)
    pltpu.sync_copy(t11_h.at[pl.ds(base, NPT16)], t1b)
    pltpu.sync_copy(dv_h.at[pl.ds(base, NPT16)], dvb)
    pltpu.sync_copy(xd_h.at[pl.ds(base, NPT16)], xdb)

    def cloop(i, _):
        sl = pl.ds(i * L, L)
        dv = dvb[sl]
        s1 = dv * (t0b[sl] + t1b[sl] + xdb[sl])
        acc0 = jnp.zeros((L,), _f32)
        acc1 = jnp.zeros((L,), _f32)
        for j in range(H1):
            hr = jnp.maximum(s1 * w1v[j] + b1v[j], 0.0)
            acc0 = acc0 + hr * w20v[j]
            acc1 = acc1 + hr * w21v[j]
        g0b[sl] = acc0 * dv
        g1b[sl] = acc1 * dv
        return 0

    lax.fori_loop(0, NPT16 // L, cloop, 0)
    pltpu.sync_copy(g0b, gd0_sp.at[pl.ds(base, NPT16)])
    pltpu.sync_copy(g1b, gd1_sp.at[pl.ds(base, NPT16)])

    def fz(i, _):
        zb[pl.ds(i * L, L)] = jnp.zeros((L,), _f32)
        return 0

    lax.fori_loop(0, ZB // L, fz, 0)
    pltpu.sync_copy(zb, a0_sp.at[pl.ds(base, ZB)])
    pltpu.sync_copy(zb, a0_sp.at[pl.ds(base + ZB, ZB)])
    pltpu.sync_copy(zb, a1_sp.at[pl.ds(base, ZB)])
    pltpu.sync_copy(zb, a1_sp.at[pl.ds(base + ZB, ZB)])

    @pl.when(c == 0)
    def _():
        pltpu.sync_copy(g0b, gd0_o.at[pl.ds(base, NPT16)])
        pltpu.sync_copy(g1b, gd1_o.at[pl.ds(base, NPT16)])

    plsc.subcore_barrier()

    def eloop(k, _):
        base = wid * EPT + k * C
        pltpu.sync_copy(src_h.at[pl.ds(base, C)], sslab)
        pltpu.sync_copy(dst_h.at[pl.ds(base, C)], dslab)
        pltpu.sync_copy(gd0_sp.at[sslab], e0b)
        pltpu.sync_copy(e0b, a0_sp.at[dslab], add=True)
        pltpu.sync_copy(gd1_sp.at[sslab], e1b)
        pltpu.sync_copy(e1b, a1_sp.at[dslab], add=True)
        return 0

    lax.fori_loop(0, NCK, eloop, 0)
    plsc.subcore_barrier()

    @pl.when(c == 0)
    def _():
        pltpu.sync_copy(a0_sp.at[pl.ds(s * NPT16, NPT16)], o00.at[pl.ds(s * NPT16, NPT16)])
        pltpu.sync_copy(a1_sp.at[pl.ds(s * NPT16, NPT16)], o01.at[pl.ds(s * NPT16, NPT16)])

    @pl.when(c == 1)
    def _():
        pltpu.sync_copy(a0_sp.at[pl.ds(s * NPT16, NPT16)], o10.at[pl.ds(s * NPT16, NPT16)])
        pltpu.sync_copy(a1_sp.at[pl.ds(s * NPT16, NPT16)], o11.at[pl.ds(s * NPT16, NPT16)])


@functools.partial(
    pl.kernel,
    out_type=[jax.ShapeDtypeStruct((128,), _f32) for _ in range(6)],
    mesh=_mesh,
    scratch_types=[
        pltpu.VMEM_SHARED((136,), _f32),
        pltpu.VMEM_SHARED((136,), _f32),
        pltpu.VMEM_SHARED((136,), _f32),
        pltpu.VMEM((NPT32,), _i32),
        pltpu.VMEM((136,), _f32),
        pltpu.VMEM((NPT32,), _f32),
        pltpu.VMEM((NPT32,), _f32),
        pltpu.VMEM((NPT32,), _f32),
        pltpu.VMEM((NPT32,), _f32),
        pltpu.VMEM((NPT32,), _f32),
        pltpu.VMEM((NPT32,), _f32),
        pltpu.VMEM((NPT32,), _f32),
        pltpu.VMEM((NPT32,), _f32),
        pltpu.VMEM((NPT32,), _f32),
        pltpu.VMEM((NPT32,), _f32),
    ],
)
def _k_pool(t00_h, t01_h, t10_h, t11_h, dv_h, g0_h, g1_h, bt_h,
            po00, po01, pc0, po10, po11, pc1,
            ps0, ps1, psc, btb, zb, onesb,
            a00b, a01b, a10b, a11b, dvb, g0b, g1b, o0b, o1b):
    c = lax.axis_index("c")
    s = lax.axis_index("s")
    wid = c * NS + s
    _fill(zb, 136, 0.0)

    def fl(i, _):
        onesb[pl.ds(i * L, L)] = jnp.full((L,), 1.0, _f32)
        return 0

    lax.fori_loop(0, NPT32 // L, fl, 0)

    @pl.when(s == 0)
    def _():
        pltpu.sync_copy(zb, ps0)
        pltpu.sync_copy(zb, ps1)
        pltpu.sync_copy(zb, psc)

    plsc.subcore_barrier()
    base = wid * NPT32
    pltpu.sync_copy(bt_h.at[pl.ds(base, NPT32)], btb)
    pltpu.sync_copy(t00_h.at[pl.ds(base, NPT32)], a00b)
    pltpu.sync_copy(t01_h.at[pl.ds(base, NPT32)], a01b)
    pltpu.sync_copy(t10_h.at[pl.ds(base, NPT32)], a10b)
    pltpu.sync_copy(t11_h.at[pl.ds(base, NPT32)], a11b)
    pltpu.sync_copy(dv_h.at[pl.ds(base, NPT32)], dvb)
    pltpu.sync_copy(g0_h.at[pl.ds(base, NPT32)], g0b)
    pltpu.sync_copy(g1_h.at[pl.ds(base, NPT32)], g1b)

    def cloop(i, _):
        sl = pl.ds(i * L, L)
        dv = dvb[sl]
        o0b[sl] = dv * (a00b[sl] + a10b[sl] + g0b[sl])
        o1b[sl] = dv * (a01b[sl] + a11b[sl] + g1b[sl])
        return 0

    lax.fori_loop(0, NPT32 // L, cloop, 0)
    pltpu.sync_copy(o0b, ps0.at[btb], add=True)
    pltpu.sync_copy(o1b, ps1.at[btb], add=True)
    pltpu.sync_copy(onesb, psc.at[btb], add=True)
    plsc.subcore_barrier()

    @pl.when(jnp.logical_and(s == 0, c == 0))
    def _():
        pltpu.sync_copy(ps0.at[pl.ds(0, 128)], po00)
        pltpu.sync_copy(ps1.at[pl.ds(0, 128)], po01)
        pltpu.sync_copy(psc.at[pl.ds(0, 128)], pc0)

    @pl.when(jnp.logical_and(s == 0, c == 1))
    def _():
        pltpu.sync_copy(ps0.at[pl.ds(0, 128)], po10)
        pltpu.sync_copy(ps1.at[pl.ds(0, 128)], po11)
        pltpu.sync_copy(psc.at[pl.ds(0, 128)], pc1)


def kernel(x, edge_index, batch, W1, b1, W2, b2):
    x_pad = jnp.concatenate([x[:, 0], jnp.zeros((NP - N,), _f32)])
    pad_idx = jnp.full((EP - E,), N, _i32)
    src1d = jnp.concatenate([edge_index[0], pad_idx])
    dst1d = jnp.concatenate([edge_index[1], pad_idx])
    bt_pad = jnp.concatenate([batch, jnp.full((NP - N,), G, _i32)])
    w1m = jnp.broadcast_to(W1.reshape(H1)[:, None], (H1, 16)).astype(_f32)
    b1m = jnp.broadcast_to(b1[:, None], (H1, 16)).astype(_f32)
    w20m = jnp.broadcast_to(W2[:, 0][:, None], (H1, 16)).astype(_f32)
    w21m = jnp.broadcast_to(W2[:, 1][:, None], (H1, 16)).astype(_f32)

    d0, d1 = _k_deg(dst1d)
    t10, t11, dinv, xd = _k_t1(src1d, dst1d, d0, d1, x_pad)
    t00, t01, t10_, t11_, gd0, gd1 = _k_t2(
        src1d, dst1d, t10, t11, dinv, xd, w1m, b1m, w20m, w21m)
    p00, p01, c0, p10, p11, c1 = _k_pool(
        t00, t01, t10_, t11_, dinv, gd0, gd1, bt_pad)

    pool = jnp.stack([p00 + p10, p01 + p11], axis=1)
    cnt = c0 + c1
    mean = pool / jnp.maximum(cnt, 1.0)[:, None] + b2[None, :]
    return jax.nn.log_softmax(mean, axis=1)


# speedup vs baseline: 138.6005x; 1.0787x over previous
import functools

import jax
import jax.numpy as jnp
from jax import lax
from jax.experimental import pallas as pl
from jax.experimental.pallas import tpu as pltpu
from jax.experimental.pallas import tpu_sc as plsc

N = 100000
E = 1600000
G = 128
H1 = 16

NC = 2
NS = 16
L = 16

NP = 102400
EP = 1605632
EPT = EP // (NC * NS)
C = 7168
NCK = EPT // C
C3 = 3584
NCK3 = EPT // C3
NPT16 = NP // NS
NPT32 = NP // (NC * NS)
ZB = 3200

_f32 = jnp.float32
_i32 = jnp.int32

_mesh = plsc.VectorSubcoreMesh(core_axis_name="c", subcore_axis_name="s")


def _rsqrt16(d):
    bits = lax.bitcast_convert_type(d, _i32)
    bits = jnp.int32(0x5F3759DF) - lax.shift_right_logical(bits, 1)
    y = lax.bitcast_convert_type(bits, _f32)
    for _ in range(3):
        y = y * (1.5 - 0.5 * d * y * y)
    return y


def _fill(ref, n, value, dtype=_f32):
    for i in range(n // L):
        ref[pl.ds(i * L, L)] = jnp.full((L,), value, dtype)


@functools.partial(
    pl.kernel,
    out_type=[jax.ShapeDtypeStruct((NP,), _f32) for _ in range(2)],
    mesh=_mesh,
    scratch_types=[
        pltpu.VMEM_SHARED((NP,), _f32),
        pltpu.VMEM((C,), _i32),
        pltpu.VMEM((C,), _i32),
        pltpu.VMEM((C,), _i32),
        pltpu.VMEM((C,), _f32),
        pltpu.VMEM((ZB,), _f32),
        pltpu.SemaphoreType.DMA,
        pltpu.SemaphoreType.DMA,
        pltpu.SemaphoreType.DMA,
        pltpu.SemaphoreType.DMA,
    ],
)
def _k_deg(dst_h, o0, o1, acc_sp, db0, db1, db2, onesb, zb, lsem, ss0, ss1, ss2):
    c = lax.axis_index("c")
    s = lax.axis_index("s")
    wid = c * NS + s

    def fl(i, _):
        onesb[pl.ds(i * L, L)] = jnp.full((L,), 1.0, _f32)
        return 0

    lax.fori_loop(0, C // L, fl, 0)

    def fz(i, _):
        zb[pl.ds(i * L, L)] = jnp.zeros((L,), _f32)
        return 0

    lax.fori_loop(0, ZB // L, fz, 0)
    pltpu.sync_copy(zb, acc_sp.at[pl.ds(s * NPT16, ZB)])
    pltpu.sync_copy(zb, acc_sp.at[pl.ds(s * NPT16 + ZB, ZB)])
    plsc.subcore_barrier()

    ssem = [ss0, ss1, ss2]
    dbuf = [db0, db1, db2]
    ldh = pltpu.async_copy(dst_h.at[pl.ds(wid * EPT, C)], dbuf[0], lsem)
    scats = []
    for k in range(NCK):
        r = k % 3
        ldh.wait()
        if k >= 2:
            scats[k - 2].wait()
        if k + 1 < NCK:
            ldh = pltpu.async_copy(
                dst_h.at[pl.ds(wid * EPT + (k + 1) * C, C)], dbuf[(k + 1) % 3], lsem)
        scats.append(pltpu.async_copy(onesb, acc_sp.at[dbuf[r]], ssem[r], add=True))
    scats[-2].wait()
    scats[-1].wait()
    plsc.subcore_barrier()

    @pl.when(c == 0)
    def _():
        pltpu.sync_copy(acc_sp.at[pl.ds(s * NPT16, NPT16)], o0.at[pl.ds(s * NPT16, NPT16)])

    @pl.when(c == 1)
    def _():
        pltpu.sync_copy(acc_sp.at[pl.ds(s * NPT16, NPT16)], o1.at[pl.ds(s * NPT16, NPT16)])


@functools.partial(
    pl.kernel,
    out_type=[jax.ShapeDtypeStruct((NP,), _f32) for _ in range(4)],
    mesh=_mesh,
    scratch_types=[
        pltpu.VMEM_SHARED((NP,), _f32),
        pltpu.VMEM_SHARED((NP,), _f32),
        pltpu.VMEM((C,), _i32),
        pltpu.VMEM((C,), _i32),
        pltpu.VMEM((C,), _i32),
        pltpu.VMEM((C,), _i32),
        pltpu.VMEM((C,), _i32),
        pltpu.VMEM((C,), _i32),
        pltpu.VMEM((ZB,), _f32),
        pltpu.VMEM((C,), _f32),
        pltpu.VMEM((C,), _f32),
        pltpu.VMEM((NPT16,), _f32),
        pltpu.VMEM((NPT16,), _f32),
        pltpu.VMEM((NPT16,), _f32),
        pltpu.VMEM((NPT16,), _f32),
        pltpu.VMEM((NPT16,), _f32),
        pltpu.SemaphoreType.DMA,
        pltpu.SemaphoreType.DMA,
        pltpu.SemaphoreType.DMA,
        pltpu.SemaphoreType.DMA,
        pltpu.SemaphoreType.DMA,
        pltpu.SemaphoreType.DMA,
    ],
)
def _k_t1(src_h, dst_h, d0_h, d1_h, x_h, t1o0, t1o1, dinv_o, xd_o,
          xd_sp, acc_sp, sl0, sl1, sl2, dl0, dl1, dl2, zb, gb0, gb1,
          d0b, d1b, xb, dvb, xdb,
          lsem_s, lsem_d, gsem, ss0, ss1, ss2):
    c = lax.axis_index("c")
    s = lax.axis_index("s")
    wid = c * NS + s
    base = s * NPT16
    pltpu.sync_copy(d0_h.at[pl.ds(base, NPT16)], d0b)
    pltpu.sync_copy(d1_h.at[pl.ds(base, NPT16)], d1b)
    pltpu.sync_copy(x_h.at[pl.ds(base, NPT16)], xb)

    def cloop(i, _):
        sl = pl.ds(i * L, L)
        deg = d0b[sl] + d1b[sl] + 1.0
        dv = _rsqrt16(deg)
        dvb[sl] = dv
        xdb[sl] = xb[sl] * dv
        return 0

    lax.fori_loop(0, NPT16 // L, cloop, 0)
    pltpu.sync_copy(xdb, xd_sp.at[pl.ds(base, NPT16)])

    def fz(i, _):
        zb[pl.ds(i * L, L)] = jnp.zeros((L,), _f32)
        return 0

    lax.fori_loop(0, ZB // L, fz, 0)
    pltpu.sync_copy(zb, acc_sp.at[pl.ds(base, ZB)])
    pltpu.sync_copy(zb, acc_sp.at[pl.ds(base + ZB, ZB)])

    @pl.when(c == 0)
    def _():
        pltpu.sync_copy(dvb, dinv_o.at[pl.ds(base, NPT16)])
        pltpu.sync_copy(xdb, xd_o.at[pl.ds(base, NPT16)])

    plsc.subcore_barrier()

    ssem = [ss0, ss1, ss2]
    sslab = [sl0, sl1, sl2]
    dslab = [dl0, dl1, dl2]
    gb = [gb0, gb1]
    eb = wid * EPT
    lh_s = pltpu.async_copy(src_h.at[pl.ds(eb, C)], sslab[0], lsem_s)
    lh_d = pltpu.async_copy(dst_h.at[pl.ds(eb, C)], dslab[0], lsem_d)
    scats = []
    for k in range(NCK):
        r = k % 3
        p = k % 2
        lh_s.wait()
        lh_d.wait()
        if k >= 2:
            scats[k - 2].wait()
        if k + 1 < NCK:
            nr = (k + 1) % 3
            lh_s = pltpu.async_copy(src_h.at[pl.ds(eb + (k + 1) * C, C)], sslab[nr], lsem_s)
            lh_d = pltpu.async_copy(dst_h.at[pl.ds(eb + (k + 1) * C, C)], dslab[nr], lsem_d)
        pltpu.async_copy(xd_sp.at[sslab[r]], gb[p], gsem).wait()
        scats.append(pltpu.async_copy(gb[p], acc_sp.at[dslab[r]], ssem[r], add=True))
    scats[-2].wait()
    scats[-1].wait()
    plsc.subcore_barrier()

    @pl.when(c == 0)
    def _():
        pltpu.sync_copy(acc_sp.at[pl.ds(s * NPT16, NPT16)], t1o0.at[pl.ds(s * NPT16, NPT16)])

    @pl.when(c == 1)
    def _():
        pltpu.sync_copy(acc_sp.at[pl.ds(s * NPT16, NPT16)], t1o1.at[pl.ds(s * NPT16, NPT16)])


@functools.partial(
    pl.kernel,
    out_type=[jax.ShapeDtypeStruct((NP,), _f32) for _ in range(6)],
    mesh=_mesh,
    scratch_types=[
        pltpu.VMEM_SHARED((NP,), _f32),
        pltpu.VMEM_SHARED((NP,), _f32),
        pltpu.VMEM_SHARED((NP,), _f32),
        pltpu.VMEM_SHARED((NP,), _f32),
        pltpu.VMEM((C3,), _i32),
        pltpu.VMEM((C3,), _i32),
        pltpu.VMEM((C3,), _i32),
        pltpu.VMEM((C3,), _i32),
        pltpu.VMEM((C3,), _i32),
        pltpu.VMEM((C3,), _i32),
        pltpu.VMEM((C3,), _f32),
        pltpu.VMEM((C3,), _f32),
        pltpu.VMEM((C3,), _f32),
        pltpu.VMEM((C3,), _f32),
        pltpu.VMEM((16, 16), _f32),
        pltpu.VMEM((16, 16), _f32),
        pltpu.VMEM((16, 16), _f32),
        pltpu.VMEM((16, 16), _f32),
        pltpu.VMEM((ZB,), _f32),
        pltpu.VMEM((NPT16,), _f32),
        pltpu.VMEM((NPT16,), _f32),
        pltpu.VMEM((NPT16,), _f32),
        pltpu.VMEM((NPT16,), _f32),
        pltpu.VMEM((NPT16,), _f32),
        pltpu.VMEM((NPT16,), _f32),
        pltpu.SemaphoreType.DMA,
        pltpu.SemaphoreType.DMA,
        pltpu.SemaphoreType.DMA,
        pltpu.SemaphoreType.DMA,
        pltpu.SemaphoreType.DMA,
        pltpu.SemaphoreType.DMA,
        pltpu.SemaphoreType.DMA,
        pltpu.SemaphoreType.DMA,
        pltpu.SemaphoreType.DMA,
        pltpu.SemaphoreType.DMA,
    ],
)
def _k_t2(src_h, dst_h, t10_h, t11_h, dv_h, xd_h, w1_h, b1_h, w20_h, w21_h,
          o00, o01, o10, o11, gd0_o, gd1_o,
          gd0_sp, gd1_sp, a0_sp, a1_sp, sl0, sl1, sl2, dl0, dl1, dl2,
          e00, e01, e10, e11,
          w1v, b1v, w20v, w21v, zb, t0b, t1b, dvb, xdb, g0b, g1b,
          lsem_s, lsem_d, gsem0, gsem1, sa0, sa1, sa2, sb0, sb1, sb2):
    c = lax.axis_index("c")
    s = lax.axis_index("s")
    wid = c * NS + s
    pltpu.sync_copy(w1_h, w1v)
    pltpu.sync_copy(b1_h, b1v)
    pltpu.sync_copy(w20_h, w20v)
    pltpu.sync_copy(w21_h, w21v)

    base = s * NPT16
    pltpu.sync_copy(t10_h.at[pl.ds(base, NPT16)], t0b)
    pltpu.sync_copy(t11_h.at[pl.ds(base, NPT16)], t1b)
    pltpu.sync_copy(dv_h.at[pl.ds(base, NPT16)], dvb)
    pltpu.sync_copy(xd_h.at[pl.ds(base, NPT16)], xdb)

    def cloop(i, _):
        sl = pl.ds(i * L, L)
        dv = dvb[sl]
        s1 = dv * (t0b[sl] + t1b[sl] + xdb[sl])
        acc0 = jnp.zeros((L,), _f32)
        acc1 = jnp.zeros((L,), _f32)
        for j in range(H1):
            hr = jnp.maximum(s1 * w1v[j] + b1v[j], 0.0)
            acc0 = acc0 + hr * w20v[j]
            acc1 = acc1 + hr * w21v[j]
        g0b[sl] = acc0 * dv
        g1b[sl] = acc1 * dv
        return 0

    lax.fori_loop(0, NPT16 // L, cloop, 0)
    pltpu.sync_copy(g0b, gd0_sp.at[pl.ds(base, NPT16)])
    pltpu.sync_copy(g1b, gd1_sp.at[pl.ds(base, NPT16)])

    def fz(i, _):
        zb[pl.ds(i * L, L)] = jnp.zeros((L,), _f32)
        return 0

    lax.fori_loop(0, ZB // L, fz, 0)
    pltpu.sync_copy(zb, a0_sp.at[pl.ds(base, ZB)])
    pltpu.sync_copy(zb, a0_sp.at[pl.ds(base + ZB, ZB)])
    pltpu.sync_copy(zb, a1_sp.at[pl.ds(base, ZB)])
    pltpu.sync_copy(zb, a1_sp.at[pl.ds(base + ZB, ZB)])

    @pl.when(c == 0)
    def _():
        pltpu.sync_copy(g0b, gd0_o.at[pl.ds(base, NPT16)])
        pltpu.sync_copy(g1b, gd1_o.at[pl.ds(base, NPT16)])

    plsc.subcore_barrier()

    sA = [sa0, sa1, sa2]
    sB = [sb0, sb1, sb2]
    sslab = [sl0, sl1, sl2]
    dslab = [dl0, dl1, dl2]
    e0b = [e00, e01]
    e1b = [e10, e11]
    eb = wid * EPT
    lh_s = pltpu.async_copy(src_h.at[pl.ds(eb, C3)], sslab[0], lsem_s)
    lh_d = pltpu.async_copy(dst_h.at[pl.ds(eb, C3)], dslab[0], lsem_d)
    scA = []
    scB = []
    for k in range(NCK3):
        r = k % 3
        p = k % 2
        lh_s.wait()
        lh_d.wait()
        if k >= 2:
            scA[k - 2].wait()
            scB[k - 2].wait()
        if k + 1 < NCK3:
            nr = (k + 1) % 3
            lh_s = pltpu.async_copy(src_h.at[pl.ds(eb + (k + 1) * C3, C3)], sslab[nr], lsem_s)
            lh_d = pltpu.async_copy(dst_h.at[pl.ds(eb + (k + 1) * C3, C3)], dslab[nr], lsem_d)
        gh0 = pltpu.async_copy(gd0_sp.at[sslab[r]], e0b[p], gsem0)
        gh1 = pltpu.async_copy(gd1_sp.at[sslab[r]], e1b[p], gsem1)
        gh0.wait()
        scA.append(pltpu.async_copy(e0b[p], a0_sp.at[dslab[r]], sA[r], add=True))
        gh1.wait()
        scB.append(pltpu.async_copy(e1b[p], a1_sp.at[dslab[r]], sB[r], add=True))
    scA[-2].wait()
    scA[-1].wait()
    scB[-2].wait()
    scB[-1].wait()
    plsc.subcore_barrier()

    @pl.when(c == 0)
    def _():
        pltpu.sync_copy(a0_sp.at[pl.ds(s * NPT16, NPT16)], o00.at[pl.ds(s * NPT16, NPT16)])
        pltpu.sync_copy(a1_sp.at[pl.ds(s * NPT16, NPT16)], o01.at[pl.ds(s * NPT16, NPT16)])

    @pl.when(c == 1)
    def _():
        pltpu.sync_copy(a0_sp.at[pl.ds(s * NPT16, NPT16)], o10.at[pl.ds(s * NPT16, NPT16)])
        pltpu.sync_copy(a1_sp.at[pl.ds(s * NPT16, NPT16)], o11.at[pl.ds(s * NPT16, NPT16)])


@functools.partial(
    pl.kernel,
    out_type=[jax.ShapeDtypeStruct((128,), _f32) for _ in range(6)],
    mesh=_mesh,
    scratch_types=[
        pltpu.VMEM_SHARED((136,), _f32),
        pltpu.VMEM_SHARED((136,), _f32),
        pltpu.VMEM_SHARED((136,), _f32),
        pltpu.VMEM((NPT32,), _i32),
        pltpu.VMEM((136,), _f32),
        pltpu.VMEM((NPT32,), _f32),
        pltpu.VMEM((NPT32,), _f32),
        pltpu.VMEM((NPT32,), _f32),
        pltpu.VMEM((NPT32,), _f32),
        pltpu.VMEM((NPT32,), _f32),
        pltpu.VMEM((NPT32,), _f32),
        pltpu.VMEM((NPT32,), _f32),
        pltpu.VMEM((NPT32,), _f32),
        pltpu.VMEM((NPT32,), _f32),
        pltpu.VMEM((NPT32,), _f32),
    ],
)
def _k_pool(t00_h, t01_h, t10_h, t11_h, dv_h, g0_h, g1_h, bt_h,
            po00, po01, pc0, po10, po11, pc1,
            ps0, ps1, psc, btb, zb, onesb,
            a00b, a01b, a10b, a11b, dvb, g0b, g1b, o0b, o1b):
    c = lax.axis_index("c")
    s = lax.axis_index("s")
    wid = c * NS + s
    _fill(zb, 136, 0.0)

    def fl(i, _):
        onesb[pl.ds(i * L, L)] = jnp.full((L,), 1.0, _f32)
        return 0

    lax.fori_loop(0, NPT32 // L, fl, 0)

    @pl.when(s == 0)
    def _():
        pltpu.sync_copy(zb, ps0)
        pltpu.sync_copy(zb, ps1)
        pltpu.sync_copy(zb, psc)

    plsc.subcore_barrier()
    base = wid * NPT32
    pltpu.sync_copy(bt_h.at[pl.ds(base, NPT32)], btb)
    pltpu.sync_copy(t00_h.at[pl.ds(base, NPT32)], a00b)
    pltpu.sync_copy(t01_h.at[pl.ds(base, NPT32)], a01b)
    pltpu.sync_copy(t10_h.at[pl.ds(base, NPT32)], a10b)
    pltpu.sync_copy(t11_h.at[pl.ds(base, NPT32)], a11b)
    pltpu.sync_copy(dv_h.at[pl.ds(base, NPT32)], dvb)
    pltpu.sync_copy(g0_h.at[pl.ds(base, NPT32)], g0b)
    pltpu.sync_copy(g1_h.at[pl.ds(base, NPT32)], g1b)

    def cloop(i, _):
        sl = pl.ds(i * L, L)
        dv = dvb[sl]
        o0b[sl] = dv * (a00b[sl] + a10b[sl] + g0b[sl])
        o1b[sl] = dv * (a01b[sl] + a11b[sl] + g1b[sl])
        return 0

    lax.fori_loop(0, NPT32 // L, cloop, 0)
    pltpu.sync_copy(o0b, ps0.at[btb], add=True)
    pltpu.sync_copy(o1b, ps1.at[btb], add=True)
    pltpu.sync_copy(onesb, psc.at[btb], add=True)
    plsc.subcore_barrier()

    @pl.when(jnp.logical_and(s == 0, c == 0))
    def _():
        pltpu.sync_copy(ps0.at[pl.ds(0, 128)], po00)
        pltpu.sync_copy(ps1.at[pl.ds(0, 128)], po01)
        pltpu.sync_copy(psc.at[pl.ds(0, 128)], pc0)

    @pl.when(jnp.logical_and(s == 0, c == 1))
    def _():
        pltpu.sync_copy(ps0.at[pl.ds(0, 128)], po10)
        pltpu.sync_copy(ps1.at[pl.ds(0, 128)], po11)
        pltpu.sync_copy(psc.at[pl.ds(0, 128)], pc1)


def kernel(x, edge_index, batch, W1, b1, W2, b2):
    x_pad = jnp.concatenate([x[:, 0], jnp.zeros((NP - N,), _f32)])
    pad_idx = jnp.full((EP - E,), N, _i32)
    src1d = jnp.concatenate([edge_index[0], pad_idx])
    dst1d = jnp.concatenate([edge_index[1], pad_idx])
    bt_pad = jnp.concatenate([batch, jnp.full((NP - N,), G, _i32)])
    w1m = jnp.broadcast_to(W1.reshape(H1)[:, None], (H1, 16)).astype(_f32)
    b1m = jnp.broadcast_to(b1[:, None], (H1, 16)).astype(_f32)
    w20m = jnp.broadcast_to(W2[:, 0][:, None], (H1, 16)).astype(_f32)
    w21m = jnp.broadcast_to(W2[:, 1][:, None], (H1, 16)).astype(_f32)

    d0, d1 = _k_deg(dst1d)
    t10, t11, dinv, xd = _k_t1(src1d, dst1d, d0, d1, x_pad)
    t00, t01, t10_, t11_, gd0, gd1 = _k_t2(
        src1d, dst1d, t10, t11, dinv, xd, w1m, b1m, w20m, w21m)
    p00, p01, c0, p10, p11, c1 = _k_pool(
        t00, t01, t10_, t11_, dinv, gd0, gd1, bt_pad)

    pool = jnp.stack([p00 + p10, p01 + p11], axis=1)
    cnt = c0 + c1
    mean = pool / jnp.maximum(cnt, 1.0)[:, None] + b2[None, :]
    return jax.nn.log_softmax(mean, axis=1)
